# Initial kernel scaffold; baseline (speedup 1.0000x reference)
import functools

import jax
import jax.numpy as jnp
from jax import lax
from jax.experimental import pallas as pl
from jax.experimental.pallas import tpu as pltpu
from jax.experimental.pallas import tpu_sc as plsc

D = 768
ACCW = 896
NC = 2
NS = 16
NW = NC * NS
GCH = 64
SCH = 32


def _mlp_body(x_ref, w1_ref, b1_ref, w2_ref, b2_ref, o_ref, *, nblk):
    i = pl.program_id(0)

    @pl.when(i < nblk)
    def _():
        x = x_ref[...]
        h = lax.dot_general(x, w1_ref[...], (((1,), (1,)), ((), ())),
                            preferred_element_type=jnp.float32)
        h = jax.nn.gelu(h + b1_ref[...], approximate=False)
        o = lax.dot_general(h, w2_ref[...], (((1,), (1,)), ((), ())),
                            preferred_element_type=jnp.float32)
        o_ref[...] = o + b2_ref[...]

    @pl.when(i >= nblk)
    def _():
        o_ref[...] = jnp.zeros_like(o_ref)


def _mlp(x, w1, b1, w2, b2, extra_zero_blocks=0, blk=256):
    n = x.shape[0]
    nblk = n // blk
    grid = (nblk + extra_zero_blocks,)
    out = pl.pallas_call(
        functools.partial(_mlp_body, nblk=nblk),
        grid=grid,
        in_specs=[
            pl.BlockSpec((blk, D), lambda i: (jnp.minimum(i, nblk - 1), 0)),
            pl.BlockSpec((D, D), lambda i: (0, 0)),
            pl.BlockSpec((1, D), lambda i: (0, 0)),
            pl.BlockSpec((D, D), lambda i: (0, 0)),
            pl.BlockSpec((1, D), lambda i: (0, 0)),
        ],
        out_specs=pl.BlockSpec((blk, D), lambda i: (i, 0)),
        out_shape=jax.ShapeDtypeStruct(((nblk + extra_zero_blocks) * blk, D),
                                       jnp.float32),
    )(x, w1, b1.reshape(1, D), w2, b2.reshape(1, D))
    return out


def _ln(x, g, b):
    mu = jnp.mean(x, axis=-1, keepdims=True)
    var = jnp.mean((x - mu) ** 2, axis=-1, keepdims=True)
    return (x - mu) * lax.rsqrt(var + 1e-5) * g + b


def _fin_jamo_body(jamo_ref, gath_ref, g_ref, b_ref, o_ref):
    x = jamo_ref[...] + gath_ref[...]
    o_ref[...] = _ln(x, g_ref[...], b_ref[...])


def _fin_jamo(jamo, gath, g, b, blk=256):
    n = jamo.shape[0]
    return pl.pallas_call(
        _fin_jamo_body,
        grid=(n // blk,),
        in_specs=[
            pl.BlockSpec((blk, D), lambda i: (i, 0)),
            pl.BlockSpec((blk, D), lambda i: (i, 0)),
            pl.BlockSpec((1, D), lambda i: (0, 0)),
            pl.BlockSpec((1, D), lambda i: (0, 0)),
        ],
        out_specs=pl.BlockSpec((blk, D), lambda i: (i, 0)),
        out_shape=jax.ShapeDtypeStruct((n, D), jnp.float32),
    )(jamo, gath, g.reshape(1, D), b.reshape(1, D))


def _fin_syll_body(acc_ref, syll_ref, g_ref, b_ref, o_ref):
    a = acc_ref[...]
    sums = a[:, :D]
    cnt = jnp.sum(a[:, D:], axis=1, keepdims=True)
    mean = jnp.where(cnt > 0, sums / jnp.maximum(cnt, 1.0), 0.0)
    x = syll_ref[...] + mean
    o_ref[...] = _ln(x, g_ref[...], b_ref[...])


def _fin_syll(acc, syll, g, b, blk=256):
    n = syll.shape[0]
    return pl.pallas_call(
        _fin_syll_body,
        grid=(n // blk,),
        in_specs=[
            pl.BlockSpec((blk, ACCW), lambda i: (i, 0)),
            pl.BlockSpec((blk, D), lambda i: (i, 0)),
            pl.BlockSpec((1, D), lambda i: (0, 0)),
            pl.BlockSpec((1, D), lambda i: (0, 0)),
        ],
        out_specs=pl.BlockSpec((blk, D), lambda i: (i, 0)),
        out_shape=jax.ShapeDtypeStruct((n, D), jnp.float32),
    )(acc, syll, g.reshape(1, D), b.reshape(1, D))


def _sc_body(table, jc, gidx, sidx, gath_out, acc_out,
             idx_g, idx_s, rows, stag, zbuf, acc, sem, *,
             per_w, nseg_core):
    c = lax.axis_index("c")
    s = lax.axis_index("s")
    wid = c * NS + s
    base = wid * per_w

    zero = jnp.zeros((16,), jnp.float32)
    one0 = jnp.where(lax.iota(jnp.int32, 16) == 0, 1.0, 0.0)
    for r in range(zbuf.shape[0]):
        for cb in range(ACCW // 16):
            zbuf[r, pl.ds(cb * 16, 16)] = zero
    for r in range(SCH):
        stag[r, pl.ds(D, 16)] = one0
        for cb in range(D // 16 + 1, ACCW // 16):
            stag[r, pl.ds(cb * 16, 16)] = zero

    rows_per_tile = nseg_core // NS
    for i in range(rows_per_tile // 16):
        pltpu.sync_copy(zbuf, acc.at[pl.ds(s * rows_per_tile + i * 16, 16)])

    for ch in range(per_w // GCH):
        off = base + ch * GCH
        pltpu.sync_copy(gidx.at[pl.ds(off, GCH)], idx_g)
        pltpu.async_copy(table.at[idx_g], rows, sem).wait()
        pltpu.sync_copy(rows, gath_out.at[pl.ds(off, GCH)])

    plsc.subcore_barrier()

    for ch in range(per_w // SCH):
        off = base + ch * SCH
        pltpu.sync_copy(sidx.at[pl.ds(off, SCH)], idx_s)
        pltpu.sync_copy(jc.at[pl.ds(off, SCH)], stag.at[:, pl.ds(0, D)])
        pltpu.sync_copy(stag, acc.at[idx_s], add=True)

    plsc.subcore_barrier()

    own = nseg_core // NS
    pltpu.sync_copy(acc.at[pl.ds(s * own, own)],
                    acc_out.at[pl.ds(c * nseg_core + s * own, own)])


def _sc_gather_segsum(table, jc, gidx, sidx, nseg):
    nj = jc.shape[0]
    per_w = nj // NW
    nseg_core = nseg // NC
    mesh = plsc.VectorSubcoreMesh(core_axis_name="c", subcore_axis_name="s",
                                  num_cores=NC, num_subcores=NS)
    kern = pl.kernel(
        functools.partial(_sc_body, per_w=per_w, nseg_core=nseg_core),
        out_type=(
            jax.ShapeDtypeStruct((nj, D), jnp.float32),
            jax.ShapeDtypeStruct((nseg, ACCW), jnp.float32),
        ),
        mesh=mesh,
        scratch_types=[
            pltpu.VMEM((GCH,), jnp.int32),
            pltpu.VMEM((SCH,), jnp.int32),
            pltpu.VMEM((GCH, D), jnp.float32),
            pltpu.VMEM((SCH, ACCW), jnp.float32),
            pltpu.VMEM((16, ACCW), jnp.float32),
            pltpu.VMEM_SHARED((nseg_core + 16, ACCW), jnp.float32),
            pltpu.SemaphoreType.DMA,
        ],
    )
    return kern(table, jc, gidx, sidx)


def kernel(jamo_features, syllable_features, syllable_indices,
           W1a, b1a, W2a, b2a, W1b, b1b, W2b, b2b, g1, beta1, g2, beta2):
    B, SJ, _ = jamo_features.shape
    _, SS, _ = syllable_features.shape
    nj = B * SJ
    nseg = B * SS

    jamo2 = jamo_features.reshape(nj, D)
    syll2 = syllable_features.reshape(nseg, D)

    table = _mlp(syll2, W1a, b1a, W2a, b2a, extra_zero_blocks=1)
    jc = _mlp(jamo2, W1b, b1b, W2b, b2b)

    si = syllable_indices.astype(jnp.int32)
    valid = (si >= 1) & (si <= SS)
    brow = (jnp.arange(B, dtype=jnp.int32) * SS)[:, None]
    gidx = jnp.where(valid, brow + si - 1, nseg).reshape(nj)
    seg_per_core = nseg // NC
    sidx = jnp.where(valid, (brow % seg_per_core) + si - 1,
                     seg_per_core).reshape(nj)

    gath, acc = _sc_gather_segsum(table, jc, gidx, sidx, nseg)

    out1 = _fin_jamo(jamo2, gath, g1, beta1)
    out2 = _fin_syll(acc, syll2, g2, beta2)
    return (out1.reshape(B, SJ, D), out2.reshape(B, SS, D))


# speedup vs baseline: 1.2018x; 1.2018x over previous
import functools

import jax
import jax.numpy as jnp
from jax import lax
from jax.experimental import pallas as pl
from jax.experimental.pallas import tpu as pltpu
from jax.experimental.pallas import tpu_sc as plsc

D = 768
ACCW = 896
NC = 2
NS = 16
NW = NC * NS
GCH = 32
SCH = 32
ZR = 8


def _mlp_body(x_ref, w1_ref, b1_ref, w2_ref, b2_ref, o_ref, *, nblk):
    i = pl.program_id(0)

    @pl.when(i < nblk)
    def _():
        x = x_ref[...]
        h = lax.dot_general(x, w1_ref[...], (((1,), (1,)), ((), ())),
                            preferred_element_type=jnp.float32)
        h = h + b1_ref[...]
        h = 0.5 * h * (1.0 + lax.erf(h * 0.7071067811865476))
        o = lax.dot_general(h, w2_ref[...], (((1,), (1,)), ((), ())),
                            preferred_element_type=jnp.float32)
        o_ref[...] = o + b2_ref[...]

    @pl.when(i >= nblk)
    def _():
        o_ref[...] = jnp.zeros_like(o_ref)


def _mlp(x, w1, b1, w2, b2, extra_zero_blocks=0, blk=256):
    n = x.shape[0]
    nblk = n // blk
    grid = (nblk + extra_zero_blocks,)
    out = pl.pallas_call(
        functools.partial(_mlp_body, nblk=nblk),
        grid=grid,
        in_specs=[
            pl.BlockSpec((blk, D), lambda i: (jnp.minimum(i, nblk - 1), 0)),
            pl.BlockSpec((D, D), lambda i: (0, 0)),
            pl.BlockSpec((1, D), lambda i: (0, 0)),
            pl.BlockSpec((D, D), lambda i: (0, 0)),
            pl.BlockSpec((1, D), lambda i: (0, 0)),
        ],
        out_specs=pl.BlockSpec((blk, D), lambda i: (i, 0)),
        out_shape=jax.ShapeDtypeStruct(((nblk + extra_zero_blocks) * blk, D),
                                       jnp.float32),
    )(x, w1, b1.reshape(1, D), w2, b2.reshape(1, D))
    return out


def _ln(x, g, b):
    mu = jnp.mean(x, axis=-1, keepdims=True)
    var = jnp.mean((x - mu) ** 2, axis=-1, keepdims=True)
    return (x - mu) * lax.rsqrt(var + 1e-5) * g + b


def _fin_jamo_body(jamo_ref, gath_ref, g_ref, b_ref, o_ref):
    x = jamo_ref[...] + gath_ref[...]
    o_ref[...] = _ln(x, g_ref[...], b_ref[...])


def _fin_jamo(jamo, gath, g, b, blk=256):
    n = jamo.shape[0]
    return pl.pallas_call(
        _fin_jamo_body,
        grid=(n // blk,),
        in_specs=[
            pl.BlockSpec((blk, D), lambda i: (i, 0)),
            pl.BlockSpec((blk, D), lambda i: (i, 0)),
            pl.BlockSpec((1, D), lambda i: (0, 0)),
            pl.BlockSpec((1, D), lambda i: (0, 0)),
        ],
        out_specs=pl.BlockSpec((blk, D), lambda i: (i, 0)),
        out_shape=jax.ShapeDtypeStruct((n, D), jnp.float32),
    )(jamo, gath, g.reshape(1, D), b.reshape(1, D))


def _fin_syll_body(acc_ref, syll_ref, g_ref, b_ref, o_ref):
    a = acc_ref[...]
    sums = a[:, :D]
    cnt = jnp.sum(a[:, D:], axis=1, keepdims=True)
    mean = jnp.where(cnt > 0, sums / jnp.maximum(cnt, 1.0), 0.0)
    x = syll_ref[...] + mean
    o_ref[...] = _ln(x, g_ref[...], b_ref[...])


def _fin_syll(acc, syll, g, b, blk=256):
    n = syll.shape[0]
    return pl.pallas_call(
        _fin_syll_body,
        grid=(n // blk,),
        in_specs=[
            pl.BlockSpec((blk, ACCW), lambda i: (i, 0)),
            pl.BlockSpec((blk, D), lambda i: (i, 0)),
            pl.BlockSpec((1, D), lambda i: (0, 0)),
            pl.BlockSpec((1, D), lambda i: (0, 0)),
        ],
        out_specs=pl.BlockSpec((blk, D), lambda i: (i, 0)),
        out_shape=jax.ShapeDtypeStruct((n, D), jnp.float32),
    )(acc, syll, g.reshape(1, D), b.reshape(1, D))


def _sc_body(table, jc, gidx, sidx, gath_out, acc_out,
             idx_g, rows, sidxv, rows2, acc, cnt_sm, sem, *,
             per_w, sj, nbatch_per_core, segs_per_tile):
    c = lax.axis_index("c")
    s = lax.axis_index("s")
    wid = c * NS + s
    base = pl.multiple_of(wid * per_w, per_w)
    i32 = jnp.int32

    for ch in range(per_w // GCH):
        off = pl.multiple_of(base + ch * GCH, GCH)
        pltpu.sync_copy(gidx.at[pl.ds(off, GCH)], idx_g)
        pltpu.async_copy(table.at[idx_g], rows, sem).wait()
        pltpu.sync_copy(rows, gath_out.at[pl.ds(off, GCH)])

    tiles_per_batch = NS // nbatch_per_core
    b = nbatch_per_core * c + s // tiles_per_batch
    band = (s % tiles_per_batch) * segs_per_tile
    lo = band
    hi = band + segs_per_tile
    bbase = pl.multiple_of(b * sj, sj)

    zero = jnp.zeros((16,), jnp.float32)

    def _zrow(i, carry):
        for cb in range(ACCW // 16):
            acc[i, pl.ds(cb * 16, 16)] = zero
        return carry

    lax.fori_loop(0, segs_per_tile, _zrow, 0)

    pltpu.sync_copy(sidx.at[pl.ds(bbase, sj)], sidxv)

    lane_iota = lax.iota(i32, 16)
    one0 = jnp.where(lane_iota == 0, 1.0, 0.0)

    cnt_sm[0] = 0
    cnt_sm[1] = 0

    def _count(i, carry):
        v = sidxv[pl.ds(i * 16, 16)]
        for j, bound in ((0, lo), (1, hi)):
            blt = v < bound
            all_lt = jnp.all(blt)
            any_lt = jnp.any(blt)

            @pl.when(all_lt)
            def _(j=j):
                cnt_sm[j] = cnt_sm[j] + 16

            @pl.when(any_lt & jnp.logical_not(all_lt))
            def _(j=j, blt=blt):
                t = cnt_sm[j]
                for lane in range(16):
                    t = t + jnp.where(
                        jnp.any(blt & (lane_iota == lane)), 1, 0).astype(i32)
                cnt_sm[j] = t
        return carry

    lax.fori_loop(0, sj // 16, _count, 0)
    start = cnt_sm[0]
    end = cnt_sm[1]
    start8 = (start // 8) * 8

    def _lane_splat(v, lane):
        idx = jnp.broadcast_to(lane.astype(i32), (16,))
        dnums = lax.GatherDimensionNumbers(
            offset_dims=(), collapsed_slice_dims=(0,), start_index_map=(0,))
        return lax.gather(v, idx[:, None], dnums, (1,),
                          mode=lax.GatherScatterMode.PROMISE_IN_BOUNDS)

    def _chunk(k, carry):
        p0 = pl.multiple_of(start8 + k * SCH, 8)
        pltpu.sync_copy(jc.at[pl.ds(bbase + p0, SCH)], rows2)
        for r in range(SCH):
            p = p0 + r

            @pl.when((p >= start) & (p < end))
            def _(p=p, r=r):
                v = sidxv[pl.ds((p // 16) * 16, 16)]
                row = _lane_splat(v, p % 16) - lo
                for cb in range(D // 16):
                    plsc.addupdate_scatter(
                        acc, [row, cb * 16 + lane_iota],
                        rows2[r, pl.ds(cb * 16, 16)])
                plsc.addupdate_scatter(acc, [row, D + lane_iota], one0)
        return carry

    nch = (end - start8 + SCH - 1) // SCH
    lax.fori_loop(0, nch, _chunk, 0)

    out_off = pl.multiple_of(
        b * (segs_per_tile * tiles_per_batch) + band, segs_per_tile)
    pltpu.sync_copy(acc, acc_out.at[pl.ds(out_off, segs_per_tile)])


def _sc_gather_segsum(table, jc, gidx, sidx, nseg, sj):
    nj = gidx.shape[0]
    per_w = nj // NW
    nbatch_per_core = (nj // sj) // NC
    segs_per_tile = nseg // (nj // sj) // (NS // nbatch_per_core)
    mesh = plsc.VectorSubcoreMesh(core_axis_name="c", subcore_axis_name="s",
                                  num_cores=NC, num_subcores=NS)
    kern = pl.kernel(
        functools.partial(_sc_body, per_w=per_w, sj=sj,
                          nbatch_per_core=nbatch_per_core,
                          segs_per_tile=segs_per_tile),
        out_type=(
            jax.ShapeDtypeStruct((nj, D), jnp.float32),
            jax.ShapeDtypeStruct((nseg, ACCW), jnp.float32),
        ),
        mesh=mesh,
        compiler_params=pltpu.CompilerParams(use_tc_tiling_on_sc=False, needs_layout_passes=False),
        scratch_types=[
            pltpu.VMEM((GCH,), jnp.int32),
            pltpu.VMEM((GCH, D), jnp.float32),
            pltpu.VMEM((sj,), jnp.int32),
            pltpu.VMEM((SCH, D), jnp.float32),
            pltpu.VMEM((segs_per_tile, ACCW), jnp.float32),
            pltpu.SMEM((2,), jnp.int32),
            pltpu.SemaphoreType.DMA,
        ],
    )
    return kern(table, jc, gidx, sidx)


def kernel(jamo_features, syllable_features, syllable_indices,
           W1a, b1a, W2a, b2a, W1b, b1b, W2b, b2b, g1, beta1, g2, beta2):
    B, SJ, _ = jamo_features.shape
    _, SS, _ = syllable_features.shape
    nj = B * SJ
    nseg = B * SS

    jamo2 = jamo_features.reshape(nj, D)
    syll2 = syllable_features.reshape(nseg, D)

    table = _mlp(syll2, W1a, b1a, W2a, b2a, extra_zero_blocks=1)
    jc = _mlp(jamo2, W1b, b1b, W2b, b2b, extra_zero_blocks=1)

    si = syllable_indices.astype(jnp.int32)
    valid = (si >= 1) & (si <= SS)
    brow = (jnp.arange(B, dtype=jnp.int32) * SS)[:, None]
    gidx = jnp.where(valid, brow + si - 1, nseg).reshape(nj)
    sidx = jnp.where(valid, si - 1, -1).reshape(nj)

    gath, acc = _sc_gather_segsum(table, jc, gidx, sidx, nseg, SJ)

    out1 = _fin_jamo(jamo2, gath, g1, beta1)
    out2 = _fin_syll(acc, syll2, g2, beta2)
    return (out1.reshape(B, SJ, D), out2.reshape(B, SS, D))
